# baseline (speedup 1.0000x reference)
import functools

import jax
import jax.numpy as jnp
from jax import lax
from jax.experimental import pallas as pl
from jax.experimental.pallas import tpu as pltpu
from jax.experimental.pallas import tpu_sc as plsc

_B = 2
_NS = 26
_ND = 13
_NF = _NS + _ND
_V = 100000
_D = 32
_L = 16
_NSLOT = _B * _NS


def _body(x_hbm, tbl_hbm, dt_hbm, out_hbm, x_v, dt_v, comb_v, stage_v, sem):
    cid = lax.axis_index("c")
    sid = lax.axis_index("s")

    @pl.when(jnp.logical_and(cid == 0, sid == 0))
    def _():
        pltpu.sync_copy(x_hbm, x_v)
        pltpu.sync_copy(dt_hbm, dt_v)

        lanes = lax.iota(jnp.int32, _L)

        flats = []
        for b in range(_B):
            for c in range(2):
                ids = x_v[pl.ds(b * _NF + c * _L, _L)].astype(jnp.int32)
                flats.append((b, c, ids))

        copies = []
        for b, c, ids in flats:
            blk = lax.shift_right_logical(ids, 3)
            for j in range(_L):
                f = c * _L + j
                if f >= _NS:
                    break
                s = b * _NS + f
                off = pl.multiple_of(blk[j] * 8, 8)
                copies.append(pltpu.async_copy(
                    tbl_hbm.at[f, pl.ds(off, 8)],
                    stage_v.at[pl.ds(s * 8, 8)], sem))

        for b in range(_B):
            dv = x_v[pl.ds(b * _NF + _NF - _L, _L)]
            for j in range(_ND):
                sc = dv[j + _L - _ND]
                for h in range(2):
                    comb_v[pl.ds((b * _NF + _NS + j) * _D + h * _L, _L)] = (
                        sc * dt_v[pl.ds(j * _D + h * _L, _L)])

        for cp in copies:
            cp.wait()

        for b, c, ids in flats:
            sub = jnp.bitwise_and(ids, 7)
            for j in range(_L):
                f = c * _L + j
                if f >= _NS:
                    break
                s = b * _NS + f
                i0 = jnp.broadcast_to(sub[j], (_L,)) + s * 8
                for h in range(2):
                    row = plsc.load_gather(stage_v, [i0, lanes + h * _L])
                    comb_v[pl.ds((b * _NF + f) * _D + h * _L, _L)] = row

        pltpu.sync_copy(comb_v, out_hbm)


_sc_call = functools.partial(
    pl.kernel,
    mesh=plsc.VectorSubcoreMesh(core_axis_name="c", subcore_axis_name="s"),
    out_type=jax.ShapeDtypeStruct((_B * _NF * _D,), jnp.float32),
    compiler_params=pltpu.CompilerParams(needs_layout_passes=False),
    scratch_types=[
        pltpu.VMEM((_B * _NF,), jnp.float32),
        pltpu.VMEM((_ND * _D,), jnp.float32),
        pltpu.VMEM((_B * _NF * _D,), jnp.float32),
        pltpu.VMEM((_NSLOT * 8, _D), jnp.float32),
        pltpu.SemaphoreType.DMA,
    ],
)(_body)


def kernel(X, sparse_tables, dense_tables):
    dt = dense_tables.reshape(_ND * _D)
    out = _sc_call(X.reshape(_B * _NF), sparse_tables, dt)
    return out.reshape(_B, _NF, _D)


# speedup vs baseline: 28.8547x; 28.8547x over previous
import functools

import jax
import jax.numpy as jnp
from jax import lax
from jax.experimental import pallas as pl
from jax.experimental.pallas import tpu as pltpu
from jax.experimental.pallas import tpu_sc as plsc

_B = 2
_NS = 26
_ND = 13
_NF = _NS + _ND
_V = 100000
_D = 32
_L = 16


def _body(x_hbm, tbl_hbm, dt_hbm, out_hbm, x_v, dt_v, comb_v, stage_v, sem):
    cid = lax.axis_index("c")
    sid = lax.axis_index("s")

    lanes = lax.iota(jnp.int32, _L)

    @pl.when(sid == 0)
    def _():
        b = cid
        ib = jnp.broadcast_to(b, (_L,))
        pltpu.sync_copy(x_hbm, x_v)
        pltpu.sync_copy(dt_hbm, dt_v)

        cols, blks = [], []
        for c in range(2):
            ids = plsc.load_gather(
                x_v, [ib, lanes + c * _L]).astype(jnp.int32)
            blks.append(jnp.bitwise_and(ids, -128))
            cols.append(jnp.bitwise_and(ids, 127))

        copies = []
        for f in range(_NS):
            off = pl.multiple_of(blks[f // _L][f % _L], 128)
            copies.append(pltpu.async_copy(
                tbl_hbm.at[f, pl.ds(0, _D), pl.ds(off, 128)],
                stage_v.at[pl.ds(f * _D, _D)], sem))

        dv = plsc.load_gather(x_v, [ib, lanes + _NF - _L])
        iz = jnp.broadcast_to(jnp.int32(0), (_L,))
        for j in range(_ND):
            sc = dv[j + _L - _ND]
            ij = jnp.broadcast_to(jnp.int32(j), (_L,))
            fcol = jnp.broadcast_to(jnp.int32(_NS + j), (_L,))
            for h in range(2):
                row = plsc.load_gather(dt_v, [ij, iz, lanes + h * _L])
                plsc.store_scatter(
                    comb_v, [lanes + h * _L, fcol], sc * row)

        for cp in copies:
            cp.wait()

        for f in range(_NS):
            i1 = jnp.broadcast_to(cols[f // _L][f % _L], (_L,))
            fcol = jnp.broadcast_to(jnp.int32(f), (_L,))
            for h in range(2):
                row = plsc.load_gather(
                    stage_v, [lanes + (f * _D + h * _L), i1])
                plsc.store_scatter(
                    comb_v, [lanes + h * _L, fcol], row)

        pltpu.sync_copy(comb_v, out_hbm.at[b])


_sc_call = functools.partial(
    pl.kernel,
    mesh=plsc.VectorSubcoreMesh(core_axis_name="c", subcore_axis_name="s"),
    out_type=jax.ShapeDtypeStruct((_B, _D, _NF), jnp.float32),
    compiler_params=pltpu.CompilerParams(needs_layout_passes=False),
    scratch_types=[
        pltpu.VMEM((_B, _NF), jnp.float32),
        pltpu.VMEM((_ND, 1, _D), jnp.float32),
        pltpu.VMEM((_D, _NF), jnp.float32),
        pltpu.VMEM((_NS * _D, 128), jnp.float32),
        pltpu.SemaphoreType.DMA,
    ],
)(_body)


def kernel(X, sparse_tables, dense_tables):
    tbl = sparse_tables.transpose(0, 2, 1)
    out = _sc_call(X, tbl, dense_tables)
    return out.transpose(0, 2, 1)


# speedup vs baseline: 33.0650x; 1.1459x over previous
import functools

import jax
import jax.numpy as jnp
from jax import lax
from jax.experimental import pallas as pl
from jax.experimental.pallas import tpu as pltpu
from jax.experimental.pallas import tpu_sc as plsc

_B = 2
_NS = 26
_ND = 13
_NF = _NS + _ND
_V = 100000
_D = 32
_L = 16
_SPT = 2
_NLT = _NS * _B // _SPT
_DT0 = _NLT


def _body(x_hbm, tbl_hbm, dt_hbm, out_hbm, x_v, dt_v, comb_v, dcomb_v,
          stage_v, sem):
    cid = lax.axis_index("c")
    sid = lax.axis_index("s")
    wid = sid * 2 + cid

    lanes = lax.iota(jnp.int32, _L)

    @pl.when(wid < _DT0 + _B)
    def _():
        pltpu.sync_copy(x_hbm, x_v)

        @pl.when(wid < _DT0)
        def _():
            s0 = wid * _SPT
            b = (wid >= _NLT // 2).astype(jnp.int32)
            f0 = s0 - b * _NS

            ids = plsc.load_gather(
                x_v, [jnp.broadcast_to(b, (_L,)), lanes + f0]
            ).astype(jnp.int32)
            blk = jnp.bitwise_and(ids, -128)
            col = jnp.bitwise_and(ids, 127)

            copies = []
            for k in range(_SPT):
                off = pl.multiple_of(blk[k], 128)
                copies.append(pltpu.async_copy(
                    tbl_hbm.at[f0 + k, pl.ds(0, _D), pl.ds(off, 128)],
                    stage_v.at[pl.ds(k * _D, _D)], sem))
            for cp in copies:
                cp.wait()

            for k in range(_SPT):
                i1 = jnp.broadcast_to(col[k], (_L,))
                for h in range(2):
                    o = k * _D + h * _L
                    comb_v[pl.ds(o, _L)] = plsc.load_gather(
                        stage_v, [lanes + o, i1])

            pltpu.sync_copy(
                comb_v,
                out_hbm.at[pl.ds(
                    pl.multiple_of(s0 * _D + b * _ND * _D, _SPT * _D),
                    _SPT * _D)])

        @pl.when(wid >= _DT0)
        def _():
            b = wid - _DT0
            pltpu.sync_copy(dt_hbm, dt_v)
            dv = plsc.load_gather(
                x_v, [jnp.broadcast_to(b, (_L,)), lanes + _NF - _L])
            iz = jnp.broadcast_to(jnp.int32(0), (_L,))
            for j in range(_ND):
                sc = dv[j + _L - _ND]
                ij = jnp.broadcast_to(jnp.int32(j), (_L,))
                for h in range(2):
                    o = j * _D + h * _L
                    dcomb_v[pl.ds(o, _L)] = sc * plsc.load_gather(
                        dt_v, [ij, iz, lanes + h * _L])

            pltpu.sync_copy(
                dcomb_v,
                out_hbm.at[pl.ds(
                    pl.multiple_of((b * _NF + _NS) * _D, _L), _ND * _D)])


_sc_call = functools.partial(
    pl.kernel,
    mesh=plsc.VectorSubcoreMesh(core_axis_name="c", subcore_axis_name="s"),
    out_type=jax.ShapeDtypeStruct((_B * _NF * _D,), jnp.float32),
    compiler_params=pltpu.CompilerParams(needs_layout_passes=False),
    scratch_types=[
        pltpu.VMEM((_B, _NF), jnp.float32),
        pltpu.VMEM((_ND, 1, _D), jnp.float32),
        pltpu.VMEM((_SPT * _D,), jnp.float32),
        pltpu.VMEM((_ND * _D,), jnp.float32),
        pltpu.VMEM((_SPT * _D, 128), jnp.float32),
        pltpu.SemaphoreType.DMA,
    ],
)(_body)


def kernel(X, sparse_tables, dense_tables):
    tbl = sparse_tables.transpose(0, 2, 1)
    out = _sc_call(X, tbl, dense_tables)
    return out.reshape(_B, _NF, _D)


# speedup vs baseline: 34.0477x; 1.0297x over previous
import functools

import jax
import jax.numpy as jnp
from jax import lax
from jax.experimental import pallas as pl
from jax.experimental.pallas import tpu as pltpu
from jax.experimental.pallas import tpu_sc as plsc

_B = 2
_NS = 26
_ND = 13
_NF = _NS + _ND
_V = 100000
_D = 32
_L = 16
_SPT = 4
_NLT = _NS * _B // _SPT
_DT0 = _NLT


def _body(x_hbm, tbl_hbm, dt_hbm, out_hbm, x_v, dt_v, comb_v, dcomb_v,
          stage_v, sem):
    sid = lax.axis_index("s")
    wid = sid

    lanes = lax.iota(jnp.int32, _L)

    @pl.when(wid < _DT0 + _B)
    def _():
        pltpu.sync_copy(x_hbm, x_v)

        @pl.when(wid < _DT0)
        def _():
            for p in range(_SPT // 2):
                s0 = wid * _SPT + 2 * p
                b = (s0 >= _NS).astype(jnp.int32)
                f0 = s0 - b * _NS

                ids = plsc.load_gather(
                    x_v, [jnp.broadcast_to(b, (_L,)), lanes + f0]
                ).astype(jnp.int32)
                blk = jnp.bitwise_and(ids, -128)
                col = jnp.bitwise_and(ids, 127)

                copies = []
                for k in range(2):
                    off = pl.multiple_of(blk[k], 128)
                    copies.append(pltpu.async_copy(
                        tbl_hbm.at[f0 + k, pl.ds(0, _D), pl.ds(off, 128)],
                        stage_v.at[pl.ds(k * _D, _D)], sem))
                for cp in copies:
                    cp.wait()

                for k in range(2):
                    i1 = jnp.broadcast_to(col[k], (_L,))
                    for h in range(2):
                        o = k * _D + h * _L
                        comb_v[pl.ds(o, _L)] = plsc.load_gather(
                            stage_v, [lanes + o, i1])

                pltpu.sync_copy(
                    comb_v,
                    out_hbm.at[pl.ds(
                        pl.multiple_of(s0 * _D + b * _ND * _D, 2 * _D),
                        2 * _D)])

        @pl.when(wid >= _DT0)
        def _():
            b = wid - _DT0
            pltpu.sync_copy(dt_hbm, dt_v)
            dv = plsc.load_gather(
                x_v, [jnp.broadcast_to(b, (_L,)), lanes + _NF - _L])
            iz = jnp.broadcast_to(jnp.int32(0), (_L,))
            for j in range(_ND):
                sc = dv[j + _L - _ND]
                ij = jnp.broadcast_to(jnp.int32(j), (_L,))
                for h in range(2):
                    o = j * _D + h * _L
                    dcomb_v[pl.ds(o, _L)] = sc * plsc.load_gather(
                        dt_v, [ij, iz, lanes + h * _L])

            pltpu.sync_copy(
                dcomb_v,
                out_hbm.at[pl.ds(
                    pl.multiple_of((b * _NF + _NS) * _D, _L), _ND * _D)])


_sc_call = functools.partial(
    pl.kernel,
    mesh=plsc.VectorSubcoreMesh(
        core_axis_name="c", subcore_axis_name="s", num_cores=1),
    out_type=jax.ShapeDtypeStruct((_B * _NF * _D,), jnp.float32),
    compiler_params=pltpu.CompilerParams(needs_layout_passes=False),
    scratch_types=[
        pltpu.VMEM((_B, _NF), jnp.float32),
        pltpu.VMEM((_ND, 1, _D), jnp.float32),
        pltpu.VMEM((2 * _D,), jnp.float32),
        pltpu.VMEM((_ND * _D,), jnp.float32),
        pltpu.VMEM((2 * _D, 128), jnp.float32),
        pltpu.SemaphoreType.DMA,
    ],
)(_body)


def kernel(X, sparse_tables, dense_tables):
    tbl = sparse_tables.transpose(0, 2, 1)
    out = _sc_call(X, tbl, dense_tables)
    return out.reshape(_B, _NF, _D)


# speedup vs baseline: 35.1777x; 1.0332x over previous
import functools

import jax
import jax.numpy as jnp
from jax import lax
from jax.experimental import pallas as pl
from jax.experimental.pallas import tpu as pltpu
from jax.experimental.pallas import tpu_sc as plsc

_B = 2
_NS = 26
_ND = 13
_NF = _NS + _ND
_V = 100000
_D = 32
_L = 16
_SPT = 4
_NLT = _NS * _B // _SPT
_DT0 = _NLT


def _body(x_hbm, tbl_hbm, dt_hbm, out_hbm, x_v, dt_v, comb_v, dcomb_v,
          stage_v, sem):
    sid = lax.axis_index("s")
    wid = sid

    lanes = lax.iota(jnp.int32, _L)

    @pl.when(wid < _DT0 + _B)
    def _():
        pltpu.sync_copy(x_hbm, x_v)

        @pl.when(wid < _DT0)
        def _():
            pairs = []
            copies = []
            for p in range(_SPT // 2):
                s0 = wid * _SPT + 2 * p
                b = (s0 >= _NS).astype(jnp.int32)
                f0 = s0 - b * _NS

                ids = plsc.load_gather(
                    x_v, [jnp.broadcast_to(b, (_L,)), lanes + f0]
                ).astype(jnp.int32)
                blk = jnp.bitwise_and(ids, -128)
                col = jnp.bitwise_and(ids, 127)
                pairs.append((s0, b, col))

                for k in range(2):
                    off = pl.multiple_of(blk[k], 128)
                    copies.append(pltpu.async_copy(
                        tbl_hbm.at[f0 + k, pl.ds(0, _D), pl.ds(off, 128)],
                        stage_v.at[pl.ds((2 * p + k) * _D, _D)], sem))
            for cp in copies:
                cp.wait()

            outs = []
            for p, (s0, b, col) in enumerate(pairs):
                for k in range(2):
                    i1 = jnp.broadcast_to(col[k], (_L,))
                    for h in range(2):
                        o = (2 * p + k) * _D + h * _L
                        comb_v[pl.ds(o, _L)] = plsc.load_gather(
                            stage_v, [lanes + o, i1])
                outs.append(pltpu.async_copy(
                    comb_v.at[pl.ds(2 * p * _D, 2 * _D)],
                    out_hbm.at[pl.ds(
                        pl.multiple_of(s0 * _D + b * _ND * _D, 2 * _D),
                        2 * _D)], sem))
            for cp in outs:
                cp.wait()

        @pl.when(wid >= _DT0)
        def _():
            b = wid - _DT0
            pltpu.sync_copy(dt_hbm, dt_v)
            dv = plsc.load_gather(
                x_v, [jnp.broadcast_to(b, (_L,)), lanes + _NF - _L])
            iz = jnp.broadcast_to(jnp.int32(0), (_L,))
            for j in range(_ND):
                sc = dv[j + _L - _ND]
                ij = jnp.broadcast_to(jnp.int32(j), (_L,))
                for h in range(2):
                    o = j * _D + h * _L
                    dcomb_v[pl.ds(o, _L)] = sc * plsc.load_gather(
                        dt_v, [ij, iz, lanes + h * _L])

            pltpu.sync_copy(
                dcomb_v,
                out_hbm.at[pl.ds(
                    pl.multiple_of((b * _NF + _NS) * _D, _L), _ND * _D)])


_sc_call = functools.partial(
    pl.kernel,
    mesh=plsc.VectorSubcoreMesh(
        core_axis_name="c", subcore_axis_name="s", num_cores=1),
    out_type=jax.ShapeDtypeStruct((_B * _NF * _D,), jnp.float32),
    compiler_params=pltpu.CompilerParams(needs_layout_passes=False),
    scratch_types=[
        pltpu.VMEM((_B, _NF), jnp.float32),
        pltpu.VMEM((_ND, 1, _D), jnp.float32),
        pltpu.VMEM((_SPT * _D,), jnp.float32),
        pltpu.VMEM((_ND * _D,), jnp.float32),
        pltpu.VMEM((_SPT * _D, 128), jnp.float32),
        pltpu.SemaphoreType.DMA,
    ],
)(_body)


def kernel(X, sparse_tables, dense_tables):
    tbl = sparse_tables.transpose(0, 2, 1)
    out = _sc_call(X, tbl, dense_tables)
    return out.reshape(_B, _NF, _D)
